# Initial kernel scaffold; baseline (speedup 1.0000x reference)
import jax
import jax.numpy as jnp
from jax.experimental import pallas as pl


def kernel(indices, values, w, V, b):
    raise NotImplementedError("write your pallas kernel here")


# speedup vs baseline: 1.3684x; 1.3684x over previous
import functools

import jax
import jax.numpy as jnp
from jax import lax
from jax.experimental import pallas as pl
from jax.experimental.pallas import tpu as pltpu
from jax.experimental.pallas import tpu_sc as plsc

L = 16
NC = 2
NS = 16
NW = NC * NS
F = 26
GC = 128


_DN = lax.GatherDimensionNumbers(
    offset_dims=(), collapsed_slice_dims=(0,), start_index_map=(0,))


def _perm(vec, idx):
    return lax.gather(vec, idx[:, None], _DN, (1,),
                      mode=lax.GatherScatterMode.PROMISE_IN_BOUNDS)


def _bcast_lane(vec, lane):
    return _perm(vec, jnp.full((L,), lane, dtype=jnp.int32))


def _lanesum(x, iota):
    for sh in (8, 4, 2, 1):
        x = x + _perm(x, jnp.bitwise_xor(iota, sh))
    return x


@functools.lru_cache(maxsize=None)
def _build(B):
    RPW = B // NW
    CH = 128
    CHN = CH * F
    NCHUNK = RPW // CH
    NG = CHN // GC

    mesh = plsc.VectorSubcoreMesh(core_axis_name="c", subcore_axis_name="s")

    @functools.partial(
        pl.kernel,
        out_type=jax.ShapeDtypeStruct((B,), jnp.float32),
        mesh=mesh,
        compiler_params=pltpu.CompilerParams(use_tc_tiling_on_sc=False),
        scratch_types=[
            pltpu.VMEM((CHN,), jnp.int32),
            pltpu.VMEM((CHN + L,), jnp.float32),
            pltpu.VMEM((CHN, L), jnp.float32),
            pltpu.VMEM((CHN + L,), jnp.float32),
            pltpu.VMEM((CH,), jnp.float32),
            pltpu.VMEM((L,), jnp.float32),
            pltpu.SemaphoreType.DMA,
            pltpu.SemaphoreType.DMA,
        ],
    )
    def fm(idx_hbm, vals_hbm, w_hbm, V_hbm, b_hbm, y_hbm,
           idxv, vvals, vrows, wrows, ybuf, bv, sem_v, sem_w):
        cid = lax.axis_index("c")
        sid = lax.axis_index("s")
        wid = sid * NC + cid
        base = wid * RPW
        pltpu.sync_copy(b_hbm, bv)
        iota = lax.iota(jnp.int32, L)
        m10 = iota < (F - L)
        fzero = jnp.zeros((L,), jnp.float32)

        def chunk(ci, carry):
            rowbase = base + ci * CH
            pltpu.sync_copy(idx_hbm.at[pl.ds(rowbase * F, CHN)], idxv)
            pltpu.sync_copy(vals_hbm.at[pl.ds(rowbase * F, CHN)],
                            vvals.at[pl.ds(0, CHN)])
            cps = []
            for j in range(NG):
                sl = pl.ds(j * GC, GC)
                cps.append(pltpu.async_copy(
                    V_hbm.at[idxv.at[sl]], vrows.at[sl, :], sem_v))
                cps.append(pltpu.async_copy(
                    w_hbm.at[idxv.at[sl]], wrows.at[sl], sem_w))
            for cp in cps:
                cp.wait()

            bvec = bv[...]

            def grp(g, c2):
                def row_body(rr, lvec):
                    r = g * L + rr
                    off = r * F
                    va = vvals[pl.ds(off, L)]
                    vb = vvals[pl.ds(off + L, L)]
                    accxv = fzero
                    accx2 = fzero
                    for f in range(F):
                        if f < L:
                            bf = _bcast_lane(va, f)
                        else:
                            bf = _bcast_lane(vb, f - L)
                        t = bf * vrows[off + f, :]
                        accxv = accxv + t
                        accx2 = accx2 + t * t
                    d = accxv * accxv - accx2
                    wa = wrows[pl.ds(off, L)]
                    wb = wrows[pl.ds(off + L, L)]
                    vbm = jnp.where(m10, vb, 0.0)
                    s = _lanesum(d, iota)
                    sumv = _lanesum(va + vbm, iota)
                    xw = _lanesum(va * wa + jnp.where(m10, vb * wb, 0.0),
                                  iota)
                    logit = xw + 0.5 * s / sumv
                    return jnp.where(iota == rr, logit, lvec)

                lvec = lax.fori_loop(0, L, row_body, fzero)
                y = 1.0 / (1.0 + jnp.exp(-(lvec + bvec)))
                ybuf[pl.ds(g * L, L)] = y
                return c2

            lax.fori_loop(0, CH // L, grp, 0)
            pltpu.sync_copy(ybuf, y_hbm.at[pl.ds(rowbase, CH)])
            return carry

        lax.fori_loop(0, NCHUNK, chunk, 0)

    return fm


def kernel(indices, values, w, V, b):
    B = indices.shape[0]
    idx_flat = indices.reshape(-1).astype(jnp.int32)
    vals_flat = values.reshape(-1).astype(jnp.float32)
    w_flat = w.reshape(-1).astype(jnp.float32)
    b16 = jnp.zeros((L,), jnp.float32) + b.reshape(-1)[0].astype(jnp.float32)
    return _build(B)(idx_flat, vals_flat, w_flat,
                     V.astype(jnp.float32), b16)
